# Initial kernel scaffold; baseline (speedup 1.0000x reference)
import jax
import jax.numpy as jnp
from jax.experimental import pallas as pl


def kernel(x, pos_emb):
    raise NotImplementedError("write your pallas kernel here")


# speedup vs baseline: 1.5015x; 1.5015x over previous
import jax
import jax.numpy as jnp
from jax.experimental import pallas as pl

_SEQ_BLOCK = 512


def _add_kernel(x_ref, pos_ref, out_ref):
    out_ref[...] = x_ref[...] + pos_ref[...]


def kernel(x, pos_emb):
    b, l, d = x.shape
    pos = pos_emb[:l]
    num_blocks = l // _SEQ_BLOCK
    return pl.pallas_call(
        _add_kernel,
        grid=(num_blocks, b),
        in_specs=[
            pl.BlockSpec((1, _SEQ_BLOCK, d), lambda i, j: (j, i, 0)),
            pl.BlockSpec((_SEQ_BLOCK, d), lambda i, j: (i, 0)),
        ],
        out_specs=pl.BlockSpec((1, _SEQ_BLOCK, d), lambda i, j: (j, i, 0)),
        out_shape=jax.ShapeDtypeStruct((b, l, d), x.dtype),
    )(x, pos)


# speedup vs baseline: 1.6661x; 1.1096x over previous
import jax
import jax.numpy as jnp
from jax.experimental import pallas as pl

_SEQ_BLOCK = 1024


def _add_kernel(x_ref, pos_ref, out_ref):
    out_ref[...] = x_ref[...] + pos_ref[...]


def kernel(x, pos_emb):
    b, l, d = x.shape
    pos = pos_emb[:l]
    num_blocks = l // _SEQ_BLOCK
    return pl.pallas_call(
        _add_kernel,
        grid=(num_blocks, b),
        in_specs=[
            pl.BlockSpec((1, _SEQ_BLOCK, d), lambda i, j: (j, i, 0)),
            pl.BlockSpec((_SEQ_BLOCK, d), lambda i, j: (i, 0)),
        ],
        out_specs=pl.BlockSpec((1, _SEQ_BLOCK, d), lambda i, j: (j, i, 0)),
        out_shape=jax.ShapeDtypeStruct((b, l, d), x.dtype),
    )(x, pos)


# speedup vs baseline: 1.7378x; 1.0430x over previous
import jax
import jax.numpy as jnp
from jax.experimental import pallas as pl

_SEQ_BLOCK = 2048


def _add_kernel(x_ref, pos_ref, out_ref):
    out_ref[...] = x_ref[...] + pos_ref[...]


def kernel(x, pos_emb):
    b, l, d = x.shape
    pos = pos_emb[:l]
    num_blocks = l // _SEQ_BLOCK
    return pl.pallas_call(
        _add_kernel,
        grid=(num_blocks, b),
        in_specs=[
            pl.BlockSpec((1, _SEQ_BLOCK, d), lambda i, j: (j, i, 0)),
            pl.BlockSpec((_SEQ_BLOCK, d), lambda i, j: (i, 0)),
        ],
        out_specs=pl.BlockSpec((1, _SEQ_BLOCK, d), lambda i, j: (j, i, 0)),
        out_shape=jax.ShapeDtypeStruct((b, l, d), x.dtype),
    )(x, pos)
